# baseline (speedup 1.0000x reference)
import jax
import jax.numpy as jnp
from jax.experimental import pallas as pl

STEPS = 256
NBINS = 128
BATCH = 32
NEUR = 4096
LANES = 128
NGRP = NEUR // LANES

_e_cache = []


def _expdraws():
    if not _e_cache:
        with jax.ensure_compile_time_eval():
            _e_cache.append(jax.block_until_ready(jax.random.exponential(
                jax.random.key(42), (NBINS, BATCH, NEUR), dtype=jnp.float32)))
    return _e_cache[0]


def _spike_kernel(u_ref, e_ref, out_ref):
    u = u_ref[...]
    rates = 250.0 * u
    scale = (1.0 / rates) * 1000.0 - 1.0

    zeros = jnp.zeros((BATCH, LANES), jnp.uint32)

    def run_phase(p):
        def body(carry):
            c, acc = carry
            acc = list(acc)
            for k in range(32 * p, 32 * p + 32):
                e = e_ref[k]
                t = e * scale
                t = t + 1.0
                c = c + t
                x = c - 1.0
                idx = jnp.clip(jnp.round(x), 0.0, 256.0).astype(jnp.int32)
                w = idx >> 5
                bitm = jnp.uint32(1) << (idx & 31).astype(jnp.uint32)
                for j in range(k >> 5, 8):
                    acc[j] = acc[j] | jnp.where(w == j, bitm, zeros)
            return c, tuple(acc)
        return body

    c = jnp.zeros((BATCH, LANES), jnp.float32)
    carry = (c, (zeros,) * 8)
    carry = run_phase(0)(carry)
    for p in range(1, NBINS // 32):
        alive = jnp.any(carry[0] < 255.5)
        carry = jax.lax.cond(alive, run_phase(p), lambda x: x, carry)
    acc = carry[1]

    shifts = jax.lax.broadcasted_iota(jnp.uint32, (32, BATCH, LANES), 0)
    one = jnp.uint32(1)
    for j in range(8):
        bits = (acc[j][None] >> shifts) & one
        out_ref[32 * j:32 * (j + 1)] = (bits != 0)


def kernel(inputs):
    e = _expdraws()
    return pl.pallas_call(
        _spike_kernel,
        grid=(NGRP,),
        in_specs=[
            pl.BlockSpec((BATCH, LANES), lambda g: (0, g)),
            pl.BlockSpec((NBINS, BATCH, LANES), lambda g: (0, 0, g)),
        ],
        out_specs=pl.BlockSpec((STEPS, BATCH, LANES), lambda g: (0, 0, g)),
        out_shape=jax.ShapeDtypeStruct((STEPS, BATCH, NEUR), jnp.bool_),
    )(inputs, e)


# speedup vs baseline: 1.0484x; 1.0484x over previous
import jax
import jax.numpy as jnp
from jax.experimental import pallas as pl

STEPS = 256
NBINS = 128
BATCH = 32
NEUR = 4096
LANES = 512
NGRP = NEUR // LANES

_e_cache = []


def _expdraws():
    if not _e_cache:
        with jax.ensure_compile_time_eval():
            _e_cache.append(jax.block_until_ready(jax.random.exponential(
                jax.random.key(42), (NBINS, BATCH, NEUR), dtype=jnp.float32)))
    return _e_cache[0]


def _spike_kernel(u_ref, e_ref, out_ref):
    u = u_ref[...]
    rates = 250.0 * u
    scale = (1.0 / rates) * 1000.0 - 1.0

    zeros = jnp.zeros((BATCH, LANES), jnp.uint32)

    def run_phase(p):
        def body(carry):
            c, acc = carry
            acc = list(acc)
            for k in range(32 * p, 32 * p + 32):
                e = e_ref[k]
                t = e * scale
                t = t + 1.0
                c = c + t
                x = c - 1.0
                idx = jnp.clip(jnp.round(x), 0.0, 256.0).astype(jnp.int32)
                w = idx >> 5
                bitm = jnp.uint32(1) << (idx & 31).astype(jnp.uint32)
                for j in range(k >> 5, 8):
                    acc[j] = acc[j] | jnp.where(w == j, bitm, zeros)
            return c, tuple(acc)
        return body

    c = jnp.zeros((BATCH, LANES), jnp.float32)
    carry = (c, (zeros,) * 8)
    carry = run_phase(0)(carry)
    for p in range(1, NBINS // 32):
        alive = jnp.any(carry[0] < 255.5)
        carry = jax.lax.cond(alive, run_phase(p), lambda x: x, carry)
    acc = carry[1]

    shifts = jax.lax.broadcasted_iota(jnp.uint32, (32, BATCH, LANES), 0)
    one = jnp.uint32(1)
    for j in range(8):
        bits = (acc[j][None] >> shifts) & one
        out_ref[32 * j:32 * (j + 1)] = (bits != 0)


def kernel(inputs):
    e = _expdraws()
    return pl.pallas_call(
        _spike_kernel,
        grid=(NGRP,),
        in_specs=[
            pl.BlockSpec((BATCH, LANES), lambda g: (0, g)),
            pl.BlockSpec((NBINS, BATCH, LANES), lambda g: (0, 0, g)),
        ],
        out_specs=pl.BlockSpec((STEPS, BATCH, LANES), lambda g: (0, 0, g)),
        out_shape=jax.ShapeDtypeStruct((STEPS, BATCH, NEUR), jnp.bool_),
    )(inputs, e)
